# baseline (speedup 1.0000x reference)
import functools

import jax
import jax.numpy as jnp
from jax import lax
from jax.experimental import pallas as pl
from jax.experimental.pallas import tpu as pltpu
from jax.experimental.pallas import tpu_sc as plsc

N = 10000
E = 320000
D = 128

NC = 2
NS = 16
NW = NC * NS
K = 80
EPW = E // NW
NCHUNK = EPW // K
NPAD = 10240
NP = 3
PR = 3456
OPAD = NP * PR
AR = PR
ZSTRIPE = AR // NS
OSTRIPE = PR // NS
ECAP = EPW + 4 * K

_sc_mesh = plsc.VectorSubcoreMesh(core_axis_name="c", subcore_axis_name="s")


@functools.partial(
    pl.kernel,
    out_type=jax.ShapeDtypeStruct((NC, NPAD), jnp.float32),
    mesh=_sc_mesh,
    scratch_types=[
        pltpu.VMEM((NCHUNK, K), jnp.int32),
        pltpu.VMEM((NCHUNK, K), jnp.float32),
        pltpu.VMEM((NPAD // NS,), jnp.float32),
        pltpu.VMEM_SHARED((NPAD,), jnp.float32),
    ],
)
def _sc_degree(col3, ew3, out, colv, ewv, zb, acc):
    c = lax.axis_index("c")
    s = lax.axis_index("s")
    w = c * NS + s
    stripe = NPAD // NS

    def zb_body(i, _):
        zb[pl.ds(i * 16, 16)] = jnp.zeros((16,), jnp.float32)
        return 0

    lax.fori_loop(0, stripe // 16, zb_body, 0)
    pltpu.sync_copy(zb, acc.at[pl.ds(s * stripe, stripe)])
    pltpu.sync_copy(col3.at[w], colv)
    pltpu.sync_copy(ew3.at[w], ewv)
    plsc.subcore_barrier()

    def chunk_body(j, _):
        pltpu.sync_copy(ewv.at[j], acc.at[colv.at[j]], add=True)
        return 0

    lax.fori_loop(0, NCHUNK, chunk_body, 0)
    plsc.subcore_barrier()
    pltpu.sync_copy(acc.at[pl.ds(s * stripe, stripe)],
                    out.at[c, pl.ds(s * stripe, stripe)])


@functools.partial(
    pl.kernel,
    out_type=jax.ShapeDtypeStruct((NC, OPAD, D), jnp.float32),
    mesh=_sc_mesh,
    compiler_params=pltpu.CompilerParams(needs_layout_passes=False),
    scratch_types=[
        pltpu.VMEM((1, EPW), jnp.int32),
        pltpu.VMEM((1, EPW), jnp.int32),
        pltpu.VMEM((1, EPW), jnp.float32),
        pltpu.VMEM((ECAP,), jnp.int32),
        pltpu.VMEM((ECAP,), jnp.int32),
        pltpu.VMEM((ECAP,), jnp.float32),
        pltpu.VMEM((1, K), jnp.int32),
        pltpu.VMEM((1, K), jnp.int32),
        pltpu.VMEM((K, D), jnp.float32),
        pltpu.VMEM((K, D), jnp.float32),
        pltpu.VMEM((ZSTRIPE // 3, D), jnp.float32),
        pltpu.VMEM_SHARED((AR, D), jnp.float32),
        pltpu.SemaphoreType.DMA,
        pltpu.SemaphoreType.DMA,
        pltpu.SemaphoreType.DMA,
        pltpu.SemaphoreType.DMA,
    ],
)
def _sc_edge_agg(g, row3, col3, ew3, out, rowf, colf, ewf, rst, cst, est,
                 colwa, colwb, gbufa, gbufb, zb, acc, gsa, gsb, ssa, ssb):
    c = lax.axis_index("c")
    s = lax.axis_index("s")
    w = c * NS + s

    def zb_body(i, _):
        r = i // (D // 16)
        v = i % (D // 16)
        zb[r, pl.ds(v * 16, 16)] = jnp.zeros((16,), jnp.float32)
        return 0

    lax.fori_loop(0, (ZSTRIPE // 3) * (D // 16), zb_body, 0)

    def clear_body(i, _):
        rst[pl.ds(i * 16, 16)] = jnp.zeros((16,), jnp.int32)
        cst[pl.ds(i * 16, 16)] = jnp.zeros((16,), jnp.int32)
        est[pl.ds(i * 16, 16)] = jnp.zeros((16,), jnp.float32)
        return 0

    lax.fori_loop(0, ECAP // 16, clear_body, 0)
    pltpu.sync_copy(row3.at[w], rowf)
    pltpu.sync_copy(col3.at[w], colf)
    pltpu.sync_copy(ew3.at[w], ewf)

    def _gather_start(j, buf, sem):
        base = pl.multiple_of(j * K, 8)
        pltpu.async_copy(g.at[rst.at[pl.ds(base, K)]], buf, sem)

    def _gather_wait(j, buf, sem):
        base = pl.multiple_of(j * K, 8)
        pltpu.make_async_copy(g.at[rst.at[pl.ds(base, K)]], buf, sem).wait()

    def _scatter_start(buf, colw, sem):
        pltpu.async_copy(buf, acc.at[colw.at[0]], sem, add=True)

    def _scatter_wait(buf, colw, sem):
        pltpu.make_async_copy(buf, acc.at[colw.at[0]], sem).wait()

    def _process(j, buf, colw):
        base = pl.multiple_of(j * K, 8)
        for eb in range(K // 16):
            colw[0, pl.ds(eb * 16, 16)] = cst[pl.ds(base + eb * 16, 16)]

        def scale_body(sb, _):
            ew16 = est[pl.ds(base + sb * 16, 16)]
            for l in range(16):
                sc = ew16[l]
                e = sb * 16 + l
                for v in range(D // 16):
                    sl = pl.ds(v * 16, 16)
                    buf[e, sl] = buf[e, sl] * sc
            return 0

        lax.fori_loop(0, K // 16, scale_body, 0)

    for p in range(NP):
        def comp_body(v, off):
            sl = pl.ds(v * 16, 16)
            rel = colf[0, sl] - p * PR
            m = (rel >= 0) & (rel < PR)
            inc = plsc.cumsum(m.astype(jnp.int32))
            dst = off + (inc - m.astype(jnp.int32))
            plsc.store_scatter(cst, [dst], rel, mask=m)
            plsc.store_scatter(rst, [dst], rowf[0, sl], mask=m)
            plsc.store_scatter(est, [dst], ewf[0, sl], mask=m)
            return off + inc[15]

        off = lax.fori_loop(0, EPW // 16, comp_body, jnp.int32(0))
        for t in range(2 * K // 16):
            est[pl.ds(off + t * 16, 16)] = jnp.zeros((16,), jnp.float32)
        nfire = (off + K - 1) // K
        nfire = jnp.maximum(nfire, 1)
        nfire = nfire + (nfire & 1)
        npairs = nfire // 2

        for z in range(3):
            pltpu.sync_copy(
                zb, acc.at[pl.ds(s * ZSTRIPE + z * (ZSTRIPE // 3),
                                 ZSTRIPE // 3)])
        plsc.subcore_barrier()

        _gather_start(0, gbufa, gsa)
        _gather_start(1, gbufb, gsb)

        def pair_body(t, _):
            a = 2 * t
            b = a + 1
            _gather_wait(a, gbufa, gsa)
            _process(a, gbufa, colwa)
            _scatter_start(gbufa, colwa, ssa)
            _gather_wait(b, gbufb, gsb)
            _process(b, gbufb, colwb)
            _scatter_start(gbufb, colwb, ssb)
            _scatter_wait(gbufa, colwa, ssa)
            _gather_start(a + 2, gbufa, gsa)
            _scatter_wait(gbufb, colwb, ssb)
            _gather_start(b + 2, gbufb, gsb)
            return 0

        lax.fori_loop(0, npairs, pair_body, 0)
        _gather_wait(nfire, gbufa, gsa)
        _gather_wait(nfire + 1, gbufb, gsb)

        plsc.subcore_barrier()
        pltpu.sync_copy(
            acc.at[pl.ds(s * OSTRIPE, OSTRIPE)],
            out.at[c, pl.ds(p * PR + s * OSTRIPE, OSTRIPE), :])
        plsc.subcore_barrier()


_NB = 1024
_GRID = (NPAD // _NB,)

_rows = pl.BlockSpec((_NB, D), lambda i: (i, 0))
_spart = pl.BlockSpec((NC, _NB, D), lambda i: (0, i, 0))
_full = pl.BlockSpec((D, D), lambda i: (0, 0))
_bias = pl.BlockSpec((1, D), lambda i: (0, 0))
_degs = pl.BlockSpec((NC, _NB), lambda i: (0, i))


def _dinv_of(degp):
    return lax.rsqrt(degp[0, :] + degp[1, :] + 1.0)[:, None]


def _mm_t(a, w):
    return lax.dot_general(a, w, (((1,), (1,)), ((), ())),
                           preferred_element_type=jnp.float32)


def _tc_pre(x_ref, w1_ref, b1_ref, wg0_ref, degp_ref, g0_ref):
    y1 = _mm_t(x_ref[...], w1_ref[...]) + b1_ref[...]
    g0_ref[...] = _dinv_of(degp_ref[...]) * _mm_t(y1, wg0_ref[...])


def _tc_mid(s3_ref, g0_ref, degp_ref, bg0_ref, wg1_ref, g1_ref):
    dinv = _dinv_of(degp_ref[...])
    y2 = dinv * (s3_ref[0] + s3_ref[1] + g0_ref[...]) + bg0_ref[...]
    g1_ref[...] = dinv * _mm_t(y2, wg1_ref[...])


def _tc_post(s3_ref, g1_ref, degp_ref, bg1_ref, w2_ref, b2_ref, o_ref):
    dinv = _dinv_of(degp_ref[...])
    y3 = dinv * (s3_ref[0] + s3_ref[1] + g1_ref[...]) + bg1_ref[...]
    o_ref[...] = jax.nn.sigmoid(_mm_t(y3, w2_ref[...]) + b2_ref[...])


_gshape = jax.ShapeDtypeStruct((NPAD, D), jnp.float32)

_tc_pre_call = pl.pallas_call(
    _tc_pre,
    grid=_GRID,
    in_specs=[_rows, _full, _bias, _full, _degs],
    out_specs=_rows,
    out_shape=_gshape,
)

_tc_mid_call = pl.pallas_call(
    _tc_mid,
    grid=_GRID,
    in_specs=[_spart, _rows, _degs, _bias, _full],
    out_specs=_rows,
    out_shape=_gshape,
)

_tc_post_call = pl.pallas_call(
    _tc_post,
    grid=_GRID,
    in_specs=[_spart, _rows, _degs, _bias, _full, _bias],
    out_specs=_rows,
    out_shape=jax.ShapeDtypeStruct((N, D), jnp.float32),
)


def kernel(x, edge_index, edge_attr, W_fc1, b_fc1, W_g0, b_g0, W_g1, b_g1,
           W_fc2, b_fc2):
    row3 = edge_index[0].astype(jnp.int32).reshape(NW, 1, EPW)
    col3 = edge_index[1].astype(jnp.int32).reshape(NW, 1, EPW)
    ew3 = edge_attr.astype(jnp.float32).reshape(NW, 1, EPW)
    col3d = edge_index[1].astype(jnp.int32).reshape(NW, NCHUNK, K)
    ew3d = edge_attr.astype(jnp.float32).reshape(NW, NCHUNK, K)

    degp = _sc_degree(col3d, ew3d)
    g0 = _tc_pre_call(x, W_fc1, b_fc1.reshape(1, D), W_g0, degp)
    s0 = _sc_edge_agg(g0, row3, col3, ew3)
    g1 = _tc_mid_call(s0, g0, degp, b_g0.reshape(1, D), W_g1)
    s1 = _sc_edge_agg(g1, row3, col3, ew3)
    return _tc_post_call(s1, g1, degp, b_g1.reshape(1, D), W_fc2,
                         b_fc2.reshape(1, D))


# speedup vs baseline: 1.9979x; 1.9979x over previous
import functools

import jax
import jax.numpy as jnp
from jax import lax
from jax.experimental import pallas as pl
from jax.experimental.pallas import tpu as pltpu
from jax.experimental.pallas import tpu_sc as plsc

N = 10000
E = 320000
D = 128

NC = 2
NS = 16
NW = NC * NS
K = 80
EPW = E // NW
NCHUNK = EPW // K
NPAD = 10240
NP = 3
PR = 3456
OPAD = NP * PR
NDUMP = 128
AR = PR + NDUMP
ZSTRIPE = AR // NS
OSTRIPE = PR // NS

_sc_mesh = plsc.VectorSubcoreMesh(core_axis_name="c", subcore_axis_name="s")


@functools.partial(
    pl.kernel,
    out_type=jax.ShapeDtypeStruct((NC, NPAD), jnp.float32),
    mesh=_sc_mesh,
    scratch_types=[
        pltpu.VMEM((NCHUNK, K), jnp.int32),
        pltpu.VMEM((NCHUNK, K), jnp.float32),
        pltpu.VMEM((NPAD // NS,), jnp.float32),
        pltpu.VMEM_SHARED((NPAD,), jnp.float32),
    ],
)
def _sc_degree(col3, ew3, out, colv, ewv, zb, acc):
    c = lax.axis_index("c")
    s = lax.axis_index("s")
    w = c * NS + s
    stripe = NPAD // NS

    def zb_body(i, _):
        zb[pl.ds(i * 16, 16)] = jnp.zeros((16,), jnp.float32)
        return 0

    lax.fori_loop(0, stripe // 16, zb_body, 0)
    pltpu.sync_copy(zb, acc.at[pl.ds(s * stripe, stripe)])
    pltpu.sync_copy(col3.at[w], colv)
    pltpu.sync_copy(ew3.at[w], ewv)
    plsc.subcore_barrier()

    def chunk_body(j, _):
        pltpu.sync_copy(ewv.at[j], acc.at[colv.at[j]], add=True)
        return 0

    lax.fori_loop(0, NCHUNK, chunk_body, 0)
    plsc.subcore_barrier()
    pltpu.sync_copy(acc.at[pl.ds(s * stripe, stripe)],
                    out.at[c, pl.ds(s * stripe, stripe)])


@functools.partial(
    pl.kernel,
    out_type=jax.ShapeDtypeStruct((NC, OPAD, D), jnp.float32),
    mesh=_sc_mesh,
    scratch_types=[
        pltpu.VMEM((NCHUNK, K), jnp.int32),
        pltpu.VMEM((NCHUNK, K), jnp.int32),
        pltpu.VMEM((NCHUNK, K), jnp.float32),
        pltpu.VMEM((1, K), jnp.int32),
        pltpu.VMEM((1, K), jnp.int32),
        pltpu.VMEM((1, K), jnp.int32),
        pltpu.VMEM((1, K), jnp.int32),
        pltpu.VMEM((K, D), jnp.float32),
        pltpu.VMEM((K, D), jnp.float32),
        pltpu.VMEM((K, D), jnp.float32),
        pltpu.VMEM((K, D), jnp.float32),
        pltpu.VMEM((ZSTRIPE // 4, D), jnp.float32),
        pltpu.VMEM_SHARED((AR, D), jnp.float32),
        pltpu.SemaphoreType.DMA,
        pltpu.SemaphoreType.DMA,
        pltpu.SemaphoreType.DMA,
        pltpu.SemaphoreType.DMA,
        pltpu.SemaphoreType.DMA,
        pltpu.SemaphoreType.DMA,
        pltpu.SemaphoreType.DMA,
        pltpu.SemaphoreType.DMA,
    ],
)
def _sc_edge_agg(g, row3, col3, ew3, out, rowv, colv, ewv, colw0, colw1,
                 colw2, colw3, gbuf0, gbuf1, gbuf2, gbuf3, zb, acc,
                 gs0, gs1, gs2, gs3, ss0, ss1, ss2, ss3):
    c = lax.axis_index("c")
    s = lax.axis_index("s")
    w = c * NS + s

    def zb_body(i, _):
        r = i // (D // 16)
        v = i % (D // 16)
        zb[r, pl.ds(v * 16, 16)] = jnp.zeros((16,), jnp.float32)
        return 0

    lax.fori_loop(0, (ZSTRIPE // 4) * (D // 16), zb_body, 0)
    pltpu.sync_copy(row3.at[w], rowv)
    pltpu.sync_copy(col3.at[w], colv)
    pltpu.sync_copy(ew3.at[w], ewv)

    def _gather_start(j, buf, sem):
        pltpu.async_copy(g.at[rowv.at[j]], buf, sem)

    def _gather_wait(j, buf, sem):
        pltpu.make_async_copy(g.at[rowv.at[j]], buf, sem).wait()

    def _scatter_start(buf, colw, sem):
        pltpu.async_copy(buf, acc.at[colw.at[0]], sem, add=True)

    def _scatter_wait(buf, colw, sem):
        pltpu.make_async_copy(buf, acc.at[colw.at[0]], sem).wait()

    for p in range(NP):

        def _process(j, buf, colw):
            def scale_body(eb, _):
                sl16 = pl.ds(eb * 16, 16)
                col16 = colv[j, sl16]
                rel = col16 - p * PR
                oob = (rel < 0) | (rel >= PR)
                dump = PR + (col16 & (NDUMP - 1))
                colw[0, sl16] = jnp.where(oob, dump, rel)
                ew16 = ewv[j, sl16]
                for l in range(16):
                    sc = ew16[l]
                    e = eb * 16 + l
                    for v in range(D // 16):
                        sl = pl.ds(v * 16, 16)
                        buf[e, sl] = buf[e, sl] * sc
                return 0

            lax.fori_loop(0, K // 16, scale_body, 0)

        for z in range(4):
            pltpu.sync_copy(
                zb, acc.at[pl.ds(s * ZSTRIPE + z * (ZSTRIPE // 4),
                                 ZSTRIPE // 4)])
        plsc.subcore_barrier()

        BUFS = [(gbuf0, colw0, gs0, ss0), (gbuf1, colw1, gs1, ss1),
                (gbuf2, colw2, gs2, ss2), (gbuf3, colw3, gs3, ss3)]

        _gather_start(0, gbuf0, gs0)
        _gather_start(1, gbuf1, gs1)

        _gather_wait(0, gbuf0, gs0)
        _process(0, gbuf0, colw0)
        _scatter_start(gbuf0, colw0, ss0)
        _gather_start(2, gbuf2, gs2)
        _gather_wait(1, gbuf1, gs1)
        _process(1, gbuf1, colw1)
        _scatter_start(gbuf1, colw1, ss1)
        _gather_start(3, gbuf3, gs3)

        def quad_body(u, _):
            j = 4 * u + 2
            for q, (buf, colw, gs, ss) in enumerate(
                    (BUFS[2], BUFS[3], BUFS[0], BUFS[1])):
                jq = j + q
                _gather_wait(jq, buf, gs)
                _process(jq, buf, colw)
                _scatter_start(buf, colw, ss)
                nbuf, ncolw, ngs, nss = BUFS[(2 + q + 2) % 4]
                _scatter_wait(nbuf, ncolw, nss)
                _gather_start(jq + 2, nbuf, ngs)
            return 0

        lax.fori_loop(0, (NCHUNK - 5) // 4, quad_body, 0)

        _gather_wait(NCHUNK - 3, gbuf2, gs2)
        _process(NCHUNK - 3, gbuf2, colw2)
        _scatter_start(gbuf2, colw2, ss2)
        _scatter_wait(gbuf0, colw0, ss0)
        _gather_start(NCHUNK - 1, gbuf0, gs0)
        _gather_wait(NCHUNK - 2, gbuf3, gs3)
        _process(NCHUNK - 2, gbuf3, colw3)
        _scatter_start(gbuf3, colw3, ss3)
        _scatter_wait(gbuf1, colw1, ss1)
        _gather_wait(NCHUNK - 1, gbuf0, gs0)
        _process(NCHUNK - 1, gbuf0, colw0)
        _scatter_start(gbuf0, colw0, ss0)
        _scatter_wait(gbuf2, colw2, ss2)
        _scatter_wait(gbuf3, colw3, ss3)
        _scatter_wait(gbuf0, colw0, ss0)

        plsc.subcore_barrier()
        pltpu.sync_copy(
            acc.at[pl.ds(s * OSTRIPE, OSTRIPE)],
            out.at[c, pl.ds(p * PR + s * OSTRIPE, OSTRIPE), :])
        plsc.subcore_barrier()


_NB = 1024
_GRID = (NPAD // _NB,)

_rows = pl.BlockSpec((_NB, D), lambda i: (i, 0))
_spart = pl.BlockSpec((NC, _NB, D), lambda i: (0, i, 0))
_full = pl.BlockSpec((D, D), lambda i: (0, 0))
_bias = pl.BlockSpec((1, D), lambda i: (0, 0))
_degs = pl.BlockSpec((NC, _NB), lambda i: (0, i))


def _dinv_of(degp):
    return lax.rsqrt(degp[0, :] + degp[1, :] + 1.0)[:, None]


def _mm_t(a, w):
    return lax.dot_general(a, w, (((1,), (1,)), ((), ())),
                           preferred_element_type=jnp.float32)


def _tc_pre(x_ref, w1_ref, b1_ref, wg0_ref, degp_ref, g0_ref):
    y1 = _mm_t(x_ref[...], w1_ref[...]) + b1_ref[...]
    g0_ref[...] = _dinv_of(degp_ref[...]) * _mm_t(y1, wg0_ref[...])


def _tc_mid(s3_ref, g0_ref, degp_ref, bg0_ref, wg1_ref, g1_ref):
    dinv = _dinv_of(degp_ref[...])
    y2 = dinv * (s3_ref[0] + s3_ref[1] + g0_ref[...]) + bg0_ref[...]
    g1_ref[...] = dinv * _mm_t(y2, wg1_ref[...])


def _tc_post(s3_ref, g1_ref, degp_ref, bg1_ref, w2_ref, b2_ref, o_ref):
    dinv = _dinv_of(degp_ref[...])
    y3 = dinv * (s3_ref[0] + s3_ref[1] + g1_ref[...]) + bg1_ref[...]
    o_ref[...] = jax.nn.sigmoid(_mm_t(y3, w2_ref[...]) + b2_ref[...])


_gshape = jax.ShapeDtypeStruct((NPAD, D), jnp.float32)

_tc_pre_call = pl.pallas_call(
    _tc_pre,
    grid=_GRID,
    in_specs=[_rows, _full, _bias, _full, _degs],
    out_specs=_rows,
    out_shape=_gshape,
)

_tc_mid_call = pl.pallas_call(
    _tc_mid,
    grid=_GRID,
    in_specs=[_spart, _rows, _degs, _bias, _full],
    out_specs=_rows,
    out_shape=_gshape,
)

_tc_post_call = pl.pallas_call(
    _tc_post,
    grid=_GRID,
    in_specs=[_spart, _rows, _degs, _bias, _full, _bias],
    out_specs=_rows,
    out_shape=jax.ShapeDtypeStruct((N, D), jnp.float32),
)


def kernel(x, edge_index, edge_attr, W_fc1, b_fc1, W_g0, b_g0, W_g1, b_g1,
           W_fc2, b_fc2):
    row3 = edge_index[0].astype(jnp.int32).reshape(NW, NCHUNK, K)
    col3 = edge_index[1].astype(jnp.int32).reshape(NW, NCHUNK, K)
    ew3 = edge_attr.astype(jnp.float32).reshape(NW, NCHUNK, K)

    degp = _sc_degree(col3, ew3)
    g0 = _tc_pre_call(x, W_fc1, b_fc1.reshape(1, D), W_g0, degp)
    s0 = _sc_edge_agg(g0, row3, col3, ew3)
    g1 = _tc_mid_call(s0, g0, degp, b_g0.reshape(1, D), W_g1)
    s1 = _sc_edge_agg(g1, row3, col3, ew3)
    return _tc_post_call(s1, g1, degp, b_g1.reshape(1, D), W_fc2,
                         b_fc2.reshape(1, D))


# speedup vs baseline: 2.7734x; 1.3882x over previous
import functools

import jax
import jax.numpy as jnp
from jax import lax
from jax.experimental import pallas as pl
from jax.experimental.pallas import tpu as pltpu
from jax.experimental.pallas import tpu_sc as plsc

N = 10000
E = 320000
D = 128

NC = 2
NS = 16
NW = NC * NS
K = 80
EPW = E // NW
NCHUNK = EPW // K
NPAD = 10240
NP = 2
PR = 5120
OPAD = NP * PR
NDUMP = 512
AR = PR + NDUMP
ZSTRIPE = AR // NS
OSTRIPE = PR // NS

_sc_mesh = plsc.VectorSubcoreMesh(core_axis_name="c", subcore_axis_name="s")


@functools.partial(
    pl.kernel,
    out_type=jax.ShapeDtypeStruct((NC, NPAD), jnp.float32),
    mesh=_sc_mesh,
    scratch_types=[
        pltpu.VMEM((NCHUNK, K), jnp.int32),
        pltpu.VMEM((NCHUNK, K), jnp.float32),
        pltpu.VMEM((NPAD // NS,), jnp.float32),
        pltpu.VMEM_SHARED((NPAD,), jnp.float32),
    ],
)
def _sc_degree(col3, ew3, out, colv, ewv, zb, acc):
    c = lax.axis_index("c")
    s = lax.axis_index("s")
    w = c * NS + s
    stripe = NPAD // NS

    def zb_body(i, _):
        zb[pl.ds(i * 16, 16)] = jnp.zeros((16,), jnp.float32)
        return 0

    lax.fori_loop(0, stripe // 16, zb_body, 0)
    pltpu.sync_copy(zb, acc.at[pl.ds(s * stripe, stripe)])
    pltpu.sync_copy(col3.at[w], colv)
    pltpu.sync_copy(ew3.at[w], ewv)
    plsc.subcore_barrier()

    def chunk_body(j, _):
        pltpu.sync_copy(ewv.at[j], acc.at[colv.at[j]], add=True)
        return 0

    lax.fori_loop(0, NCHUNK, chunk_body, 0)
    plsc.subcore_barrier()
    pltpu.sync_copy(acc.at[pl.ds(s * stripe, stripe)],
                    out.at[c, pl.ds(s * stripe, stripe)])


@functools.partial(
    pl.kernel,
    out_type=jax.ShapeDtypeStruct((NC, OPAD, D), jnp.float32),
    mesh=_sc_mesh,
    scratch_types=[
        pltpu.VMEM((EPW,), jnp.int32),
        pltpu.VMEM((EPW,), jnp.int32),
        pltpu.VMEM((EPW,), jnp.float32),
        pltpu.VMEM((1, K), jnp.int32),
        pltpu.VMEM((1, K), jnp.int32),
        pltpu.VMEM((1, K), jnp.int32),
        pltpu.VMEM((1, K), jnp.int32),
        pltpu.VMEM((K, D), jnp.float32),
        pltpu.VMEM((K, D), jnp.float32),
        pltpu.VMEM((K, D), jnp.float32),
        pltpu.VMEM((K, D), jnp.float32),
        pltpu.VMEM((ZSTRIPE // 4, D), jnp.float32),
        pltpu.VMEM_SHARED((AR, D), jnp.float32),
        pltpu.SemaphoreType.DMA,
        pltpu.SemaphoreType.DMA,
        pltpu.SemaphoreType.DMA,
        pltpu.SemaphoreType.DMA,
        pltpu.SemaphoreType.DMA,
        pltpu.SemaphoreType.DMA,
        pltpu.SemaphoreType.DMA,
        pltpu.SemaphoreType.DMA,
    ],
)
def _sc_edge_agg(g, row3, col3, ew3, out, rowv, colv, ewv, colw0, colw1,
                 colw2, colw3, gbuf0, gbuf1, gbuf2, gbuf3, zb, acc,
                 gs0, gs1, gs2, gs3, ss0, ss1, ss2, ss3):
    c = lax.axis_index("c")
    s = lax.axis_index("s")
    w = c * NS + s

    def zb_body(i, _):
        r = i // (D // 16)
        v = i % (D // 16)
        zb[r, pl.ds(v * 16, 16)] = jnp.zeros((16,), jnp.float32)
        return 0

    lax.fori_loop(0, (ZSTRIPE // 4) * (D // 16), zb_body, 0)
    base_w = pl.multiple_of(w * EPW, 8)
    pltpu.sync_copy(row3.at[pl.ds(base_w, EPW)], rowv)
    pltpu.sync_copy(col3.at[pl.ds(base_w, EPW)], colv)
    pltpu.sync_copy(ew3.at[pl.ds(base_w, EPW)], ewv)

    def _gather_start(j, buf, sem):
        base = pl.multiple_of(j * K, 8)
        pltpu.async_copy(g.at[rowv.at[pl.ds(base, K)]], buf, sem)

    def _gather_wait(j, buf, sem):
        base = pl.multiple_of(j * K, 8)
        pltpu.make_async_copy(g.at[rowv.at[pl.ds(base, K)]], buf, sem).wait()

    def _scatter_start(buf, colw, sem):
        pltpu.async_copy(buf, acc.at[colw.at[0]], sem, add=True)

    def _scatter_wait(buf, colw, sem):
        pltpu.make_async_copy(buf, acc.at[colw.at[0]], sem).wait()

    for p in range(NP):

        def _process(j, buf, colw):
            jbase = pl.multiple_of(j * K, 8)

            def scale_body(eb, _):
                sl16 = pl.ds(jbase + eb * 16, 16)
                col16 = colv[sl16]
                rel = col16 - p * PR
                oob = (rel < 0) | (rel >= PR)
                dump = PR + (col16 & (NDUMP - 1))
                colw[0, pl.ds(eb * 16, 16)] = jnp.where(oob, dump, rel)
                ew16 = ewv[sl16]
                for l in range(16):
                    sc = ew16[l]
                    e = eb * 16 + l
                    for v in range(D // 16):
                        sl = pl.ds(v * 16, 16)
                        buf[e, sl] = buf[e, sl] * sc
                return 0

            lax.fori_loop(0, K // 16, scale_body, 0)

        for z in range(4):
            pltpu.sync_copy(
                zb, acc.at[pl.ds(s * ZSTRIPE + z * (ZSTRIPE // 4),
                                 ZSTRIPE // 4)])
        plsc.subcore_barrier()

        BUFS = [(gbuf0, colw0, gs0, ss0), (gbuf1, colw1, gs1, ss1),
                (gbuf2, colw2, gs2, ss2), (gbuf3, colw3, gs3, ss3)]

        _gather_start(0, gbuf0, gs0)
        _gather_start(1, gbuf1, gs1)

        _gather_wait(0, gbuf0, gs0)
        _process(0, gbuf0, colw0)
        _scatter_start(gbuf0, colw0, ss0)
        _gather_start(2, gbuf2, gs2)
        _gather_wait(1, gbuf1, gs1)
        _process(1, gbuf1, colw1)
        _scatter_start(gbuf1, colw1, ss1)
        _gather_start(3, gbuf3, gs3)

        def quad_body(u, _):
            j = 4 * u + 2
            for q, (buf, colw, gs, ss) in enumerate(
                    (BUFS[2], BUFS[3], BUFS[0], BUFS[1])):
                jq = j + q
                _gather_wait(jq, buf, gs)
                _process(jq, buf, colw)
                _scatter_start(buf, colw, ss)
                nbuf, ncolw, ngs, nss = BUFS[(2 + q + 2) % 4]
                _scatter_wait(nbuf, ncolw, nss)
                _gather_start(jq + 2, nbuf, ngs)
            return 0

        lax.fori_loop(0, (NCHUNK - 5) // 4, quad_body, 0)

        _gather_wait(NCHUNK - 3, gbuf2, gs2)
        _process(NCHUNK - 3, gbuf2, colw2)
        _scatter_start(gbuf2, colw2, ss2)
        _scatter_wait(gbuf0, colw0, ss0)
        _gather_start(NCHUNK - 1, gbuf0, gs0)
        _gather_wait(NCHUNK - 2, gbuf3, gs3)
        _process(NCHUNK - 2, gbuf3, colw3)
        _scatter_start(gbuf3, colw3, ss3)
        _scatter_wait(gbuf1, colw1, ss1)
        _gather_wait(NCHUNK - 1, gbuf0, gs0)
        _process(NCHUNK - 1, gbuf0, colw0)
        _scatter_start(gbuf0, colw0, ss0)
        _scatter_wait(gbuf2, colw2, ss2)
        _scatter_wait(gbuf3, colw3, ss3)
        _scatter_wait(gbuf0, colw0, ss0)

        plsc.subcore_barrier()
        pltpu.sync_copy(
            acc.at[pl.ds(s * OSTRIPE, OSTRIPE)],
            out.at[c, pl.ds(p * PR + s * OSTRIPE, OSTRIPE), :])
        plsc.subcore_barrier()


_NB = 1024
_GRID = (NPAD // _NB,)

_rows = pl.BlockSpec((_NB, D), lambda i: (i, 0))
_spart = pl.BlockSpec((NC, _NB, D), lambda i: (0, i, 0))
_full = pl.BlockSpec((D, D), lambda i: (0, 0))
_bias = pl.BlockSpec((1, D), lambda i: (0, 0))
_degs = pl.BlockSpec((NC, _NB), lambda i: (0, i))


def _dinv_of(degp):
    return lax.rsqrt(degp[0, :] + degp[1, :] + 1.0)[:, None]


def _mm_t(a, w):
    return lax.dot_general(a, w, (((1,), (1,)), ((), ())),
                           preferred_element_type=jnp.float32)


def _tc_pre(x_ref, w1_ref, b1_ref, wg0_ref, degp_ref, g0_ref):
    y1 = _mm_t(x_ref[...], w1_ref[...]) + b1_ref[...]
    g0_ref[...] = _dinv_of(degp_ref[...]) * _mm_t(y1, wg0_ref[...])


def _tc_mid(s3_ref, g0_ref, degp_ref, bg0_ref, wg1_ref, g1_ref):
    dinv = _dinv_of(degp_ref[...])
    y2 = dinv * (s3_ref[0] + s3_ref[1] + g0_ref[...]) + bg0_ref[...]
    g1_ref[...] = dinv * _mm_t(y2, wg1_ref[...])


def _tc_post(s3_ref, g1_ref, degp_ref, bg1_ref, w2_ref, b2_ref, o_ref):
    dinv = _dinv_of(degp_ref[...])
    y3 = dinv * (s3_ref[0] + s3_ref[1] + g1_ref[...]) + bg1_ref[...]
    o_ref[...] = jax.nn.sigmoid(_mm_t(y3, w2_ref[...]) + b2_ref[...])


_gshape = jax.ShapeDtypeStruct((NPAD, D), jnp.float32)

_tc_pre_call = pl.pallas_call(
    _tc_pre,
    grid=_GRID,
    in_specs=[_rows, _full, _bias, _full, _degs],
    out_specs=_rows,
    out_shape=_gshape,
)

_tc_mid_call = pl.pallas_call(
    _tc_mid,
    grid=_GRID,
    in_specs=[_spart, _rows, _degs, _bias, _full],
    out_specs=_rows,
    out_shape=_gshape,
)

_tc_post_call = pl.pallas_call(
    _tc_post,
    grid=_GRID,
    in_specs=[_spart, _rows, _degs, _bias, _full, _bias],
    out_specs=_rows,
    out_shape=jax.ShapeDtypeStruct((N, D), jnp.float32),
)


def kernel(x, edge_index, edge_attr, W_fc1, b_fc1, W_g0, b_g0, W_g1, b_g1,
           W_fc2, b_fc2):
    row1 = edge_index[0].astype(jnp.int32)
    col1 = edge_index[1].astype(jnp.int32)
    ew1 = edge_attr.astype(jnp.float32)
    col3d = col1.reshape(NW, NCHUNK, K)
    ew3d = ew1.reshape(NW, NCHUNK, K)

    degp = _sc_degree(col3d, ew3d)
    g0 = _tc_pre_call(x, W_fc1, b_fc1.reshape(1, D), W_g0, degp)
    s0 = _sc_edge_agg(g0, row1, col1, ew1)
    g1 = _tc_mid_call(s0, g0, degp, b_g0.reshape(1, D), W_g1)
    s1 = _sc_edge_agg(g1, row1, col1, ew1)
    return _tc_post_call(s1, g1, degp, b_g1.reshape(1, D), W_fc2,
                         b_fc2.reshape(1, D))
